# baseline (speedup 1.0000x reference)
import jax
import jax.numpy as jnp
from jax import lax
from jax.experimental import pallas as pl
from jax.experimental.pallas import tpu as pltpu, tpu_sc as plsc

_VOCAB = 1000000
_EMB = 32
_B = 4096
_L = 50
_NC = 2
_NS = 16
_NW = _NC * _NS
_TOTAL = _B * _L
_PER_W = _TOTAL // _NW
_CHUNK = 128
_NCH = _PER_W // _CHUNK
_CPL = _B // _CHUNK
_GRP = _CHUNK // 16
_VR = _VOCAB // 4
_QT = _VOCAB // 128
_QPW = 245

_mesh = plsc.VectorSubcoreMesh(
    core_axis_name="c", subcore_axis_name="s", num_cores=_NC, num_subcores=_NS
)


def _relayout_body(tabT_hbm, out_hbm, in_v, tail_v, tv_v, isem0, isem1):
    wid = lax.axis_index("s") * _NC + lax.axis_index("c")
    j16 = lax.iota(jnp.int32, 16)

    def _fire(q, b, sem):
        pltpu.async_copy(
            tabT_hbm.at[:, pl.ds(q * _CHUNK, _CHUNK)], in_v.at[b], sem
        )

    def _drain(b, sem):
        pltpu.make_async_copy(
            tabT_hbm.at[:, pl.ds(0, _CHUNK)], in_v.at[b], sem
        ).wait()

    def _transpose_out(src, q, ng):
        for g in range(ng):
            row16 = (j16 + g * 16) >> 2
            colb16 = ((j16 + g * 16) & 3) << 5
            for c in range(_EMB):
                val = src[c, pl.ds(g * 16, 16)]
                plsc.store_scatter(tv_v, [row16, colb16 + c], val)
        pltpu.sync_copy(
            tv_v.at[pl.ds(0, 4 * ng)], out_hbm.at[pl.ds(q * 32, 4 * ng)]
        )

    _fire(wid, 0, isem0)

    @pl.loop(0, _QPW, step=2)
    def _loop(k0):
        qa = wid + k0 * _NW
        q0 = wid + (k0 + 1) * _NW

        @pl.when(q0 < _QT)
        def _():
            _fire(q0, 1, isem1)

        @pl.when(qa < _QT)
        def _():
            _drain(0, isem0)
            _transpose_out(in_v.at[0], qa, _GRP)

        q1 = wid + (k0 + 2) * _NW

        @pl.when(q1 < _QT)
        def _():
            _fire(q1, 0, isem0)

        @pl.when(q0 < _QT)
        def _():
            _drain(1, isem1)
            _transpose_out(in_v.at[1], q0, _GRP)

    @pl.when(wid == 0)
    def _():
        pltpu.sync_copy(tabT_hbm.at[:, pl.ds(_QT * _CHUNK, 64)], tail_v)
        _transpose_out(tail_v, _QT, _GRP // 2)


_relayout = pl.kernel(
    _relayout_body,
    out_type=jax.ShapeDtypeStruct((_VR, 128), jnp.float32),
    mesh=_mesh,
    scratch_types=[
        pltpu.VMEM((2, _EMB, _CHUNK), jnp.float32),
        pltpu.VMEM((_EMB, 64), jnp.float32),
        pltpu.VMEM((_EMB, 128), jnp.float32),
        pltpu.SemaphoreType.DMA,
        pltpu.SemaphoreType.DMA,
    ],
    compiler_params=pltpu.CompilerParams(
        use_tc_tiling_on_sc=True, needs_layout_passes=False
    ),
)


def _gather_body(vrow_hbm, coloff_hbm, table_hbm, out_hbm,
                 vrow_v, coloff_v, big_v, out_v, gsem0, gsem1):
    wid = lax.axis_index("s") * _NC + lax.axis_index("c")
    base = wid * _PER_W
    pltpu.sync_copy(vrow_hbm.at[pl.ds(base, _PER_W)], vrow_v)
    pltpu.sync_copy(coloff_hbm.at[pl.ds(base, _PER_W)], coloff_v)

    iota = lax.iota(jnp.int32, 16)

    def _fire(j, b, sem):
        pltpu.async_copy(
            table_hbm.at[vrow_v.at[pl.ds(j * _CHUNK, _CHUNK)]],
            big_v.at[b], sem,
        )

    def _drain(b, sem):
        pltpu.make_async_copy(
            table_hbm.at[pl.ds(0, _CHUNK)], big_v.at[b], sem
        ).wait()

    def _extract_write(j, b):
        for g in range(_GRP):
            row16 = iota + g * 16
            col16 = coloff_v[pl.ds(j * _CHUNK + g * 16, 16)]
            for c in range(_EMB):
                val = plsc.load_gather(big_v.at[b], [row16, col16 + c])
                out_v[c, pl.ds(g * 16, 16)] = val
        gch = wid * _NCH + j
        l = gch // _CPL
        b0 = (gch % _CPL) * _CHUNK
        pltpu.sync_copy(out_v, out_hbm.at[l, :, pl.ds(b0, _CHUNK)])

    _fire(0, 0, gsem0)

    @pl.loop(0, _NCH, step=2)
    def _loop(j0):
        _fire(j0 + 1, 1, gsem1)
        _drain(0, gsem0)
        _extract_write(j0, 0)

        @pl.when(j0 + 2 < _NCH)
        def _():
            _fire(j0 + 2, 0, gsem0)

        _drain(1, gsem1)
        _extract_write(j0 + 1, 1)


_gather = pl.kernel(
    _gather_body,
    out_type=jax.ShapeDtypeStruct((_L, _EMB, _B), jnp.float32),
    mesh=_mesh,
    scratch_types=[
        pltpu.VMEM((_PER_W,), jnp.int32),
        pltpu.VMEM((_PER_W,), jnp.int32),
        pltpu.VMEM((2, _CHUNK, 128), jnp.float32),
        pltpu.VMEM((_EMB, _CHUNK), jnp.float32),
        pltpu.SemaphoreType.DMA,
        pltpu.SemaphoreType.DMA,
    ],
    compiler_params=pltpu.CompilerParams(
        use_tc_tiling_on_sc=True, needs_layout_passes=False
    ),
)


def kernel(unk_inputs, table):
    idx = jnp.transpose(unk_inputs).reshape(-1)
    vrow = idx >> 2
    coloff = (idx & 3) << 5
    table128 = _relayout(jnp.transpose(table))
    out = _gather(vrow, coloff, table128)
    return jnp.transpose(out, (0, 2, 1))


# speedup vs baseline: 1.3311x; 1.3311x over previous
import jax
import jax.numpy as jnp
from jax import lax
from jax.experimental import pallas as pl
from jax.experimental.pallas import tpu as pltpu, tpu_sc as plsc

_VOCAB = 1000000
_EMB = 32
_B = 4096
_L = 50
_NC = 2
_NS = 16
_NW = _NC * _NS
_TOTAL = _B * _L
_PER_W = _TOTAL // _NW
_CHUNK = 128
_NCH = _PER_W // _CHUNK
_CPL = _B // _CHUNK
_GRP = _CHUNK // 16
_VR = _VOCAB // 4
_QT = _VOCAB // 128
_QPW = 245

_mesh = plsc.VectorSubcoreMesh(
    core_axis_name="c", subcore_axis_name="s", num_cores=_NC, num_subcores=_NS
)


def _relayout_body(tabT_hbm, out_hbm, in_v, tail_v, tv_v, isem0, isem1):
    wid = lax.axis_index("s") * _NC + lax.axis_index("c")
    j16 = lax.iota(jnp.int32, 16)

    def _fire(q, b, sem):
        pltpu.async_copy(
            tabT_hbm.at[:, pl.ds(q * _CHUNK, _CHUNK)], in_v.at[b], sem
        )

    def _drain(b, sem):
        pltpu.make_async_copy(
            tabT_hbm.at[:, pl.ds(0, _CHUNK)], in_v.at[b], sem
        ).wait()

    def _transpose_out(src, q, ng):
        for g in range(ng):
            row16 = (j16 + g * 16) >> 2
            colb16 = ((j16 + g * 16) & 3) << 5
            for c0 in range(0, _EMB, 8):
                vals = [src[c0 + d, pl.ds(g * 16, 16)] for d in range(8)]
                for d in range(8):
                    plsc.store_scatter(tv_v, [row16, colb16 + (c0 + d)], vals[d])
        pltpu.sync_copy(
            tv_v.at[pl.ds(0, 4 * ng)], out_hbm.at[pl.ds(q * 32, 4 * ng)]
        )

    _fire(wid, 0, isem0)

    @pl.loop(0, _QPW, step=2)
    def _loop(k0):
        qa = wid + k0 * _NW
        q0 = wid + (k0 + 1) * _NW

        @pl.when(q0 < _QT)
        def _():
            _fire(q0, 1, isem1)

        @pl.when(qa < _QT)
        def _():
            _drain(0, isem0)
            _transpose_out(in_v.at[0], qa, _GRP)

        q1 = wid + (k0 + 2) * _NW

        @pl.when(q1 < _QT)
        def _():
            _fire(q1, 0, isem0)

        @pl.when(q0 < _QT)
        def _():
            _drain(1, isem1)
            _transpose_out(in_v.at[1], q0, _GRP)

    @pl.when(wid == 0)
    def _():
        pltpu.sync_copy(tabT_hbm.at[:, pl.ds(_QT * _CHUNK, 64)], tail_v)
        _transpose_out(tail_v, _QT, _GRP // 2)


_relayout = pl.kernel(
    _relayout_body,
    out_type=jax.ShapeDtypeStruct((_VR, 128), jnp.float32),
    mesh=_mesh,
    scratch_types=[
        pltpu.VMEM((2, _EMB, _CHUNK), jnp.float32),
        pltpu.VMEM((_EMB, 64), jnp.float32),
        pltpu.VMEM((_EMB, 128), jnp.float32),
        pltpu.SemaphoreType.DMA,
        pltpu.SemaphoreType.DMA,
    ],
    compiler_params=pltpu.CompilerParams(
        use_tc_tiling_on_sc=True, needs_layout_passes=False
    ),
)


def _gather_body(vrow_hbm, coloff_hbm, table_hbm, out_hbm,
                 vrow_v, coloff_v, big_v, out_v, gsem0, gsem1):
    wid = lax.axis_index("s") * _NC + lax.axis_index("c")
    base = wid * _PER_W
    pltpu.sync_copy(vrow_hbm.at[pl.ds(base, _PER_W)], vrow_v)
    pltpu.sync_copy(coloff_hbm.at[pl.ds(base, _PER_W)], coloff_v)

    iota = lax.iota(jnp.int32, 16)

    def _fire(j, b, sem):
        pltpu.async_copy(
            table_hbm.at[vrow_v.at[pl.ds(j * _CHUNK, _CHUNK)]],
            big_v.at[b], sem,
        )

    def _drain(b, sem):
        pltpu.make_async_copy(
            table_hbm.at[pl.ds(0, _CHUNK)], big_v.at[b], sem
        ).wait()

    def _extract_write(j, b):
        for g in range(_GRP):
            row16 = iota + g * 16
            col16 = coloff_v[pl.ds(j * _CHUNK + g * 16, 16)]
            for c0 in range(0, _EMB, 8):
                vals = [
                    plsc.load_gather(big_v.at[b], [row16, col16 + (c0 + d)])
                    for d in range(8)
                ]
                for d in range(8):
                    out_v[c0 + d, pl.ds(g * 16, 16)] = vals[d]
        gch = wid * _NCH + j
        l = gch // _CPL
        b0 = (gch % _CPL) * _CHUNK
        pltpu.sync_copy(out_v, out_hbm.at[l, :, pl.ds(b0, _CHUNK)])

    _fire(0, 0, gsem0)

    @pl.loop(0, _NCH, step=2)
    def _loop(j0):
        _fire(j0 + 1, 1, gsem1)
        _drain(0, gsem0)
        _extract_write(j0, 0)

        @pl.when(j0 + 2 < _NCH)
        def _():
            _fire(j0 + 2, 0, gsem0)

        _drain(1, gsem1)
        _extract_write(j0 + 1, 1)


_gather = pl.kernel(
    _gather_body,
    out_type=jax.ShapeDtypeStruct((_L, _EMB, _B), jnp.float32),
    mesh=_mesh,
    scratch_types=[
        pltpu.VMEM((_PER_W,), jnp.int32),
        pltpu.VMEM((_PER_W,), jnp.int32),
        pltpu.VMEM((2, _CHUNK, 128), jnp.float32),
        pltpu.VMEM((_EMB, _CHUNK), jnp.float32),
        pltpu.SemaphoreType.DMA,
        pltpu.SemaphoreType.DMA,
    ],
    compiler_params=pltpu.CompilerParams(
        use_tc_tiling_on_sc=True, needs_layout_passes=False
    ),
)


def kernel(unk_inputs, table):
    idx = jnp.transpose(unk_inputs).reshape(-1)
    vrow = idx >> 2
    coloff = (idx & 3) << 5
    table128 = _relayout(jnp.transpose(table))
    out = _gather(vrow, coloff, table128)
    return jnp.transpose(out, (0, 2, 1))
